# baseline (speedup 1.0000x reference)
import functools

import jax
import jax.numpy as jnp
from jax import lax
from jax.experimental import pallas as pl
from jax.experimental.pallas import tpu as pltpu
from jax.experimental.pallas import tpu_sc as plsc

BATCH = 16384
NFEAT = 64
NIDX = 20
NUM_CORES = 2
NUM_SUBCORES = 16
NUM_WORKERS = NUM_CORES * NUM_SUBCORES
ROWS_PER_TILE = BATCH // NUM_WORKERS
GROUPS = ROWS_PER_TILE // 16


def _sc_body(y_hbm, stds_hbm, means_hbm, a_hbm, l_hbm, e_hbm, slack_hbm,
             out_hbm, y_v, w_v, stds_v, means_v, a_v, l_v, e_v, slack_v, sem):
    wid = lax.axis_index("s") * NUM_CORES + lax.axis_index("c")
    base = wid * ROWS_PER_TILE

    row_cp = pltpu.async_copy(y_hbm.at[pl.ds(base, ROWS_PER_TILE)], y_v, sem)
    pltpu.sync_copy(stds_hbm, stds_v)
    pltpu.sync_copy(means_hbm, means_v)
    pltpu.sync_copy(a_hbm, a_v)
    pltpu.sync_copy(l_hbm, l_v)
    pltpu.sync_copy(e_hbm, e_v)
    pltpu.sync_copy(slack_hbm, slack_v)

    lane = lax.iota(jnp.int32, 16)
    zero = jnp.zeros((16,), jnp.float32)
    for q in range(NFEAT // 16):
        w_v[pl.ds(q * 16, 16)] = zero

    tail_idx = jnp.minimum(lane + 16, NIDX - 1)
    tail_valid = (lane + 16) < NIDX
    c_parts = zero
    for idx_ref, sign in ((a_v, 1.0), (l_v, -1.0), (e_v, -1.0)):
        idx0 = idx_ref[pl.ds(0, 16)]
        plsc.addupdate_scatter(w_v, [idx0],
                               plsc.load_gather(stds_v, [idx0]) * sign)
        c_parts = c_parts + plsc.load_gather(means_v, [idx0]) * sign
        idx1 = plsc.load_gather(idx_ref, [tail_idx])
        sv1 = plsc.load_gather(stds_v, [idx1])
        mv1 = plsc.load_gather(means_v, [idx1])
        plsc.addupdate_scatter(w_v, [idx1],
                               jnp.where(tail_valid, sv1 * sign, 0.0),
                               mask=tail_valid)
        c_parts = c_parts + jnp.where(tail_valid, mv1 * sign, 0.0)
    c = jnp.sum(c_parts)

    sl = slack_v[...]
    inv_std_sl = 1.0 / plsc.load_gather(stds_v, [sl])

    row_cp.wait()

    blk = 2

    @plsc.parallel_loop(jnp.int32(0), jnp.int32(GROUPS // blk), jnp.int32(1),
                        unroll=4)
    def block_body(b):
        row0 = b.astype(jnp.int32) * (16 * blk)
        rows = [row0 + 16 * gi + lane for gi in range(blk)]
        accs = [jnp.full((16,), c, jnp.float32) for _ in range(blk)]
        for j in range(NFEAT):
            jv = jnp.bitwise_and(lane + j, NFEAT - 1)
            wj = plsc.load_gather(w_v, [jv])
            for gi in range(blk):
                yv = plsc.load_gather(y_v, [rows[gi], jv])
                accs[gi] = accs[gi] + wj * yv
        for gi in range(blk):
            plsc.addupdate_scatter(y_v, [rows[gi], sl], accs[gi] * inv_std_sl)

    pltpu.sync_copy(y_v, out_hbm.at[pl.ds(base, ROWS_PER_TILE)])


@functools.partial(jax.jit, static_argnames=())
def _sc_call(y, stds32, means32, a_idx, l_idx, e_idx, slack_vec):
    mesh = plsc.VectorSubcoreMesh(core_axis_name="c", subcore_axis_name="s",
                                  num_cores=NUM_CORES,
                                  num_subcores=NUM_SUBCORES)
    return pl.kernel(
        _sc_body,
        out_type=jax.ShapeDtypeStruct((BATCH, NFEAT), jnp.float32),
        mesh=mesh,
        compiler_params=pltpu.CompilerParams(needs_layout_passes=False,
                                             use_tc_tiling_on_sc=True),
        scratch_types=[
            pltpu.VMEM((ROWS_PER_TILE, NFEAT), jnp.float32),
            pltpu.VMEM((NFEAT,), jnp.float32),
            pltpu.VMEM((NFEAT,), jnp.float32),
            pltpu.VMEM((NFEAT,), jnp.float32),
            pltpu.VMEM((NIDX,), jnp.int32),
            pltpu.VMEM((NIDX,), jnp.int32),
            pltpu.VMEM((NIDX,), jnp.int32),
            pltpu.VMEM((16,), jnp.int32),
            pltpu.SemaphoreType.DMA,
        ],
    )(y, stds32, means32, a_idx, l_idx, e_idx, slack_vec)


def kernel(y, means, stds, asset_idx, liability_idx, equity_idx, slack_idx):
    stds32 = stds.astype(jnp.float32)
    means32 = means.astype(jnp.float32)
    a_idx = asset_idx.astype(jnp.int32)
    l_idx = liability_idx.astype(jnp.int32)
    e_idx = equity_idx.astype(jnp.int32)
    slack_vec = jnp.full((16,), jnp.asarray(slack_idx, jnp.int32), jnp.int32)
    return _sc_call(y, stds32, means32, a_idx, l_idx, e_idx, slack_vec)


# speedup vs baseline: 1.0132x; 1.0132x over previous
import functools

import jax
import jax.numpy as jnp
from jax import lax
from jax.experimental import pallas as pl
from jax.experimental.pallas import tpu as pltpu
from jax.experimental.pallas import tpu_sc as plsc

BATCH = 16384
NFEAT = 64
NIDX = 20
NUM_CORES = 2
NUM_SUBCORES = 16
NUM_WORKERS = NUM_CORES * NUM_SUBCORES
ROWS_PER_TILE = BATCH // NUM_WORKERS
GROUPS = ROWS_PER_TILE // 16


def _sc_body(y_hbm, stds_hbm, means_hbm, a_hbm, l_hbm, e_hbm, slack_hbm,
             out_hbm, y_v, w_v, stds_v, means_v, a_v, l_v, e_v, slack_v, sem):
    wid = lax.axis_index("s") * NUM_CORES + lax.axis_index("c")
    base = wid * ROWS_PER_TILE

    row_cp = pltpu.async_copy(y_hbm.at[pl.ds(base, ROWS_PER_TILE)], y_v, sem)
    pltpu.sync_copy(stds_hbm, stds_v)
    pltpu.sync_copy(means_hbm, means_v)
    pltpu.sync_copy(a_hbm, a_v)
    pltpu.sync_copy(l_hbm, l_v)
    pltpu.sync_copy(e_hbm, e_v)
    pltpu.sync_copy(slack_hbm, slack_v)

    lane = lax.iota(jnp.int32, 16)
    zero = jnp.zeros((16,), jnp.float32)
    for q in range(NFEAT // 16):
        w_v[pl.ds(q * 16, 16)] = zero

    tail_idx = jnp.minimum(lane + 16, NIDX - 1)
    tail_valid = (lane + 16) < NIDX
    c_parts = zero
    for idx_ref, sign in ((a_v, 1.0), (l_v, -1.0), (e_v, -1.0)):
        idx0 = idx_ref[pl.ds(0, 16)]
        plsc.addupdate_scatter(w_v, [idx0],
                               plsc.load_gather(stds_v, [idx0]) * sign)
        c_parts = c_parts + plsc.load_gather(means_v, [idx0]) * sign
        idx1 = plsc.load_gather(idx_ref, [tail_idx])
        sv1 = plsc.load_gather(stds_v, [idx1])
        mv1 = plsc.load_gather(means_v, [idx1])
        plsc.addupdate_scatter(w_v, [idx1],
                               jnp.where(tail_valid, sv1 * sign, 0.0),
                               mask=tail_valid)
        c_parts = c_parts + jnp.where(tail_valid, mv1 * sign, 0.0)
    c = jnp.sum(c_parts)

    sl = slack_v[...]
    inv_std_sl = 1.0 / plsc.load_gather(stds_v, [sl])

    row_cp.wait()

    blk = 4

    def block_body(b, carry):
        row0 = b * (16 * blk)
        rows = [row0 + 16 * gi + lane for gi in range(blk)]
        accs = [jnp.full((16,), c, jnp.float32) for _ in range(blk)]
        for j in range(NFEAT):
            jv = jnp.bitwise_and(lane + j, NFEAT - 1)
            wj = plsc.load_gather(w_v, [jv])
            for gi in range(blk):
                yv = plsc.load_gather(y_v, [rows[gi], jv])
                accs[gi] = accs[gi] + wj * yv
        for gi in range(blk):
            plsc.addupdate_scatter(y_v, [rows[gi], sl], accs[gi] * inv_std_sl)
        return carry

    lax.fori_loop(jnp.int32(0), jnp.int32(GROUPS // blk), block_body,
                  jnp.int32(0))

    pltpu.sync_copy(y_v, out_hbm.at[pl.ds(base, ROWS_PER_TILE)])


@functools.partial(jax.jit, static_argnames=())
def _sc_call(y, stds32, means32, a_idx, l_idx, e_idx, slack_vec):
    mesh = plsc.VectorSubcoreMesh(core_axis_name="c", subcore_axis_name="s",
                                  num_cores=NUM_CORES,
                                  num_subcores=NUM_SUBCORES)
    return pl.kernel(
        _sc_body,
        out_type=jax.ShapeDtypeStruct((BATCH, NFEAT), jnp.float32),
        mesh=mesh,
        compiler_params=pltpu.CompilerParams(needs_layout_passes=False,
                                             use_tc_tiling_on_sc=True),
        scratch_types=[
            pltpu.VMEM((ROWS_PER_TILE, NFEAT), jnp.float32),
            pltpu.VMEM((NFEAT,), jnp.float32),
            pltpu.VMEM((NFEAT,), jnp.float32),
            pltpu.VMEM((NFEAT,), jnp.float32),
            pltpu.VMEM((NIDX,), jnp.int32),
            pltpu.VMEM((NIDX,), jnp.int32),
            pltpu.VMEM((NIDX,), jnp.int32),
            pltpu.VMEM((16,), jnp.int32),
            pltpu.SemaphoreType.DMA,
        ],
    )(y, stds32, means32, a_idx, l_idx, e_idx, slack_vec)


def kernel(y, means, stds, asset_idx, liability_idx, equity_idx, slack_idx):
    stds32 = stds.astype(jnp.float32)
    means32 = means.astype(jnp.float32)
    a_idx = asset_idx.astype(jnp.int32)
    l_idx = liability_idx.astype(jnp.int32)
    e_idx = equity_idx.astype(jnp.int32)
    slack_vec = jnp.full((16,), jnp.asarray(slack_idx, jnp.int32), jnp.int32)
    return _sc_call(y, stds32, means32, a_idx, l_idx, e_idx, slack_vec)
